# baseline (device time: 21314 ns/iter reference)
import jax
import jax.numpy as jnp
from jax import lax
from jax.experimental import pallas as pl
from jax.experimental.pallas import tpu as pltpu

N_DEV = 4


def kernel(x, k, Wp):
    B, S, C = x.shape
    KT = k.shape[0]
    W = 4
    HS = S // W

    def body(x_ref, k_ref, w_ref, out_ref,
             rs_src, rs_buf, ag_src,
             rs_send, rs_recv, ag_send, ag_recv):
        my = lax.axis_index("i")

        barrier = pltpu.get_barrier_semaphore()
        for d in range(N_DEV):
            @pl.when(my != d)
            def _():
                pl.semaphore_signal(
                    barrier, inc=1,
                    device_id=(d,), device_id_type=pl.DeviceIdType.MESH,
                )

        w = w_ref[...].astype(jnp.bfloat16)
        kt_rows = [k_ref[t, :].reshape(1, C).astype(jnp.bfloat16)
                   for t in range(KT)]

        def compute_batch(b):
            xb = x_ref[b].astype(jnp.bfloat16)
            conv = xb * kt_rows[KT - 1]
            for t in range(KT - 1):
                shift = KT - 1 - t
                shifted = jnp.concatenate(
                    [jnp.zeros((shift, C), xb.dtype), xb[: S - shift, :]],
                    axis=0,
                )
                conv = conv + shifted * kt_rows[t]
            a = conv / (1.0 + jnp.exp(-conv))
            return jnp.dot(a, w, preferred_element_type=jnp.float32)

        for j in range(1, N_DEV):
            b = (my + j) % N_DEV
            pb = compute_batch(b)
            rs_src[j - 1, :, :] = pb.astype(jnp.bfloat16)
            if j == 1:
                pl.semaphore_wait(barrier, N_DEV - 1)
            for h in range(W):
                pltpu.make_async_remote_copy(
                    src_ref=rs_src.at[j - 1, pl.ds(h * HS, HS), :],
                    dst_ref=rs_buf.at[j - 1, pl.ds(h * HS, HS), :],
                    send_sem=rs_send.at[W * (j - 1) + h],
                    recv_sem=rs_recv.at[W * (j - 1) + h],
                    device_id=(b,),
                    device_id_type=pl.DeviceIdType.MESH,
                ).start()

        own = compute_batch(my)

        ag_rdmas = []
        for h in range(W):
            red = own[h * HS:(h + 1) * HS, :]
            for slot in (0, 2, 1):
                pltpu.make_async_remote_copy(
                    src_ref=rs_src.at[slot, pl.ds(h * HS, HS), :],
                    dst_ref=rs_buf.at[slot, pl.ds(h * HS, HS), :],
                    send_sem=rs_send.at[W * slot + h],
                    recv_sem=rs_recv.at[W * slot + h],
                    device_id=(0,), device_id_type=pl.DeviceIdType.MESH,
                ).wait_recv()
                red = red + rs_buf[slot, h * HS:(h + 1) * HS, :].astype(
                    jnp.float32)
            red_bf = red.astype(jnp.bfloat16)
            ag_src[pl.ds(h * HS, HS), :] = red_bf
            out_ref[pl.ds(my, 1), pl.ds(h * HS, HS), :] = red_bf.reshape(
                1, HS, C)
            for delta in range(1, N_DEV):
                tgt = (my + delta) % N_DEV
                rdma = pltpu.make_async_remote_copy(
                    src_ref=ag_src.at[pl.ds(h * HS, HS), :],
                    dst_ref=out_ref.at[my, pl.ds(h * HS, HS), :],
                    send_sem=ag_send.at[W * (delta - 1) + h],
                    recv_sem=ag_recv.at[W * (delta - 1) + h],
                    device_id=(tgt,),
                    device_id_type=pl.DeviceIdType.MESH,
                )
                rdma.start()
                ag_rdmas.append(rdma)

        for rdma in ag_rdmas:
            rdma.wait_recv()
        for rdma in ag_rdmas:
            rdma.wait_send()
        for s in range(W * (N_DEV - 1)):
            pltpu.make_async_remote_copy(
                src_ref=rs_src.at[0, pl.ds(0, HS), :],
                dst_ref=rs_buf.at[0, pl.ds(0, HS), :],
                send_sem=rs_send.at[s], recv_sem=rs_recv.at[0],
                device_id=(0,), device_id_type=pl.DeviceIdType.MESH,
            ).wait_send()

    return pl.pallas_call(
        body,
        out_shape=jax.ShapeDtypeStruct((B, S, C), jnp.bfloat16),
        in_specs=[
            pl.BlockSpec(memory_space=pltpu.VMEM),
            pl.BlockSpec(memory_space=pltpu.VMEM),
            pl.BlockSpec(memory_space=pltpu.VMEM),
        ],
        out_specs=pl.BlockSpec(memory_space=pltpu.VMEM),
        scratch_shapes=[
            pltpu.VMEM((N_DEV - 1, S, C), jnp.bfloat16),
            pltpu.VMEM((N_DEV - 1, S, C), jnp.bfloat16),
            pltpu.VMEM((S, C), jnp.bfloat16),
            pltpu.SemaphoreType.DMA((4 * (N_DEV - 1),)),
            pltpu.SemaphoreType.DMA((4 * (N_DEV - 1),)),
            pltpu.SemaphoreType.DMA((4 * (N_DEV - 1),)),
            pltpu.SemaphoreType.DMA((4 * (N_DEV - 1),)),
        ],
        compiler_params=pltpu.CompilerParams(collective_id=0),
    )(x, k, Wp)


# device time: 21046 ns/iter; 1.0127x vs baseline; 1.0127x over previous
import jax
import jax.numpy as jnp
from jax import lax
from jax.experimental import pallas as pl
from jax.experimental.pallas import tpu as pltpu

N_DEV = 4


def kernel(x, k, Wp):
    B, S, C = x.shape
    KT = k.shape[0]
    W = 4
    HS = S // W

    def body(x_ref, k_ref, w_ref, out_ref,
             rs_src, rs_buf, ag_src,
             rs_send, rs_recv, ag_send, ag_recv):
        my = lax.axis_index("i")

        barrier = pltpu.get_barrier_semaphore()
        for d in range(N_DEV):
            @pl.when(my != d)
            def _():
                pl.semaphore_signal(
                    barrier, inc=1,
                    device_id=(d,), device_id_type=pl.DeviceIdType.MESH,
                )

        w = w_ref[...].astype(jnp.bfloat16)
        kt_rows = [k_ref[t, :].reshape(1, C) for t in range(KT)]

        def compute_batch(b):
            xb = x_ref[b]
            conv = xb * kt_rows[KT - 1]
            for t in range(KT - 1):
                shift = KT - 1 - t
                shifted = jnp.concatenate(
                    [jnp.zeros((shift, C), xb.dtype), xb[: S - shift, :]],
                    axis=0,
                )
                conv = conv + shifted * kt_rows[t]
            a = conv / (1.0 + jnp.exp(-conv))
            return jnp.dot(a.astype(jnp.bfloat16), w,
                           preferred_element_type=jnp.float32)

        for j in range(1, N_DEV):
            b = (my + j) % N_DEV
            pb = compute_batch(b)
            rs_src[j - 1, :, :] = pb.astype(jnp.bfloat16)
            if j == 1:
                pl.semaphore_wait(barrier, N_DEV - 1)
            for h in range(W):
                pltpu.make_async_remote_copy(
                    src_ref=rs_src.at[j - 1, pl.ds(h * HS, HS), :],
                    dst_ref=rs_buf.at[j - 1, pl.ds(h * HS, HS), :],
                    send_sem=rs_send.at[W * (j - 1) + h],
                    recv_sem=rs_recv.at[W * (j - 1) + h],
                    device_id=(b,),
                    device_id_type=pl.DeviceIdType.MESH,
                ).start()

        own = compute_batch(my)

        ag_rdmas = []
        for h in range(W):
            red = own[h * HS:(h + 1) * HS, :]
            for slot in (0, 2, 1):
                pltpu.make_async_remote_copy(
                    src_ref=rs_src.at[slot, pl.ds(h * HS, HS), :],
                    dst_ref=rs_buf.at[slot, pl.ds(h * HS, HS), :],
                    send_sem=rs_send.at[W * slot + h],
                    recv_sem=rs_recv.at[W * slot + h],
                    device_id=(0,), device_id_type=pl.DeviceIdType.MESH,
                ).wait_recv()
                red = red + rs_buf[slot, h * HS:(h + 1) * HS, :].astype(
                    jnp.float32)
            red_bf = red.astype(jnp.bfloat16)
            ag_src[pl.ds(h * HS, HS), :] = red_bf
            out_ref[pl.ds(my, 1), pl.ds(h * HS, HS), :] = red_bf.reshape(
                1, HS, C)
            for delta in range(1, N_DEV):
                tgt = (my + delta) % N_DEV
                rdma = pltpu.make_async_remote_copy(
                    src_ref=ag_src.at[pl.ds(h * HS, HS), :],
                    dst_ref=out_ref.at[my, pl.ds(h * HS, HS), :],
                    send_sem=ag_send.at[W * (delta - 1) + h],
                    recv_sem=ag_recv.at[W * (delta - 1) + h],
                    device_id=(tgt,),
                    device_id_type=pl.DeviceIdType.MESH,
                )
                rdma.start()
                ag_rdmas.append(rdma)

        for rdma in ag_rdmas:
            rdma.wait_recv()
        for rdma in ag_rdmas:
            rdma.wait_send()
        for s in range(W * (N_DEV - 1)):
            pltpu.make_async_remote_copy(
                src_ref=rs_src.at[0, pl.ds(0, HS), :],
                dst_ref=rs_buf.at[0, pl.ds(0, HS), :],
                send_sem=rs_send.at[s], recv_sem=rs_recv.at[0],
                device_id=(0,), device_id_type=pl.DeviceIdType.MESH,
            ).wait_send()

    return pl.pallas_call(
        body,
        out_shape=jax.ShapeDtypeStruct((B, S, C), jnp.bfloat16),
        in_specs=[
            pl.BlockSpec(memory_space=pltpu.VMEM),
            pl.BlockSpec(memory_space=pltpu.VMEM),
            pl.BlockSpec(memory_space=pltpu.VMEM),
        ],
        out_specs=pl.BlockSpec(memory_space=pltpu.VMEM),
        scratch_shapes=[
            pltpu.VMEM((N_DEV - 1, S, C), jnp.bfloat16),
            pltpu.VMEM((N_DEV - 1, S, C), jnp.bfloat16),
            pltpu.VMEM((S, C), jnp.bfloat16),
            pltpu.SemaphoreType.DMA((4 * (N_DEV - 1),)),
            pltpu.SemaphoreType.DMA((4 * (N_DEV - 1),)),
            pltpu.SemaphoreType.DMA((4 * (N_DEV - 1),)),
            pltpu.SemaphoreType.DMA((4 * (N_DEV - 1),)),
        ],
        compiler_params=pltpu.CompilerParams(collective_id=0),
    )(x, k, Wp)
